# Initial kernel scaffold; baseline (speedup 1.0000x reference)
import jax
import jax.numpy as jnp
from jax.experimental import pallas as pl


def kernel(token_seq, itype_seq, opnd_type_seq, reg_id_seq, reg_r_seq, reg_w_seq, eflags_seq, token_table, itype_table, opnd_type_table, reg_id_table, reg_r_table, reg_w_table, eflags_table, pe):
    raise NotImplementedError("write your pallas kernel here")


# speedup vs baseline: 6.3781x; 6.3781x over previous
import functools

import jax
import jax.numpy as jnp
from jax import lax
from jax.experimental import pallas as pl
from jax.experimental.pallas import tpu as pltpu
from jax.experimental.pallas import tpu_sc as plsc

B, S, D = 1024, 200, 128
AUX_V = 1000
N = B * S
NW = 32
C = 40
PER_W = N // NW
NCH = PER_W // C
G = 8
NG = NCH // G
IDX_ROWS = N // C
LANES = 16
VPR = D // LANES


def _sc_body(tok_i, it_i, op_i, rid_i, rr_i, rw_i, ef_i,
             tok_t, a0, a1, a2, a3, a4, a5, pe,
             out,
             bufs, pe_v, idx_v, sem):
    cid = lax.axis_index("c")
    sid = lax.axis_index("s")
    wid = sid * 2 + cid

    tables = (tok_t, a0, a1, a2, a3, a4, a5)
    pltpu.sync_copy(pe, pe_v)

    idx_hbm = (tok_i, it_i, op_i, rid_i, rr_i, rw_i, ef_i)
    base_row = wid * NCH

    def group_body(g, carry):
        for j in range(7):
            pltpu.sync_copy(idx_hbm[j].at[pl.ds(base_row + g * G, G)],
                            idx_v.at[j])

        def chunk_body(cc, carry2):
            c = g * G + cc
            handles = []
            for j in range(7):
                handles.append(pltpu.async_copy(
                    tables[j].at[idx_v.at[j, cc]], bufs.at[j], sem))
            for h in handles:
                h.wait()

            pe_base = lax.rem(c * C, 200)

            def row_body(r, carry3):
                for v in range(VPR):
                    sl = pl.ds(v * LANES, LANES)
                    acc = bufs[0, r, sl]
                    for j in range(1, 7):
                        acc = acc + bufs[j, r, sl]
                    acc = acc + pe_v[pe_base + r, sl]
                    bufs[0, r, sl] = acc
                return carry3

            lax.fori_loop(0, C, row_body, 0, unroll=2)
            pltpu.sync_copy(bufs.at[0],
                            out.at[pl.ds((wid * NCH + c) * C, C)])
            return carry2

        lax.fori_loop(0, G, chunk_body, 0)
        return carry

    lax.fori_loop(0, NG, group_body, 0)


def kernel(token_seq, itype_seq, opnd_type_seq, reg_id_seq, reg_r_seq,
           reg_w_seq, eflags_seq, token_table, itype_table, opnd_type_table,
           reg_id_table, reg_r_table, reg_w_table, eflags_table, pe):
    idx2d = [a.reshape(IDX_ROWS, C) for a in
             (token_seq, itype_seq, opnd_type_seq, reg_id_seq, reg_r_seq,
              reg_w_seq, eflags_seq)]
    pe_t = pe[:S]

    mesh = plsc.VectorSubcoreMesh(core_axis_name="c", subcore_axis_name="s")
    run = functools.partial(
        pl.kernel,
        out_type=jax.ShapeDtypeStruct((N, D), jnp.float32),
        mesh=mesh,
        scratch_types=[
            pltpu.VMEM((7, C, D), jnp.float32),
            pltpu.VMEM((S, D), jnp.float32),
            pltpu.VMEM((7, G, C), jnp.int32),
            pltpu.SemaphoreType.DMA,
        ],
    )(_sc_body)

    out = run(*idx2d, token_table, itype_table, opnd_type_table,
              reg_id_table, reg_r_table, reg_w_table, eflags_table, pe_t)
    return out.reshape(B, S, D)
